# baseline (speedup 1.0000x reference)
import functools
import jax
import jax.numpy as jnp
from jax import lax
from jax.experimental import pallas as pl
from jax.experimental.pallas import tpu as pltpu
from jax.experimental.pallas import tpu_sc as plsc

N = 2048
DIM = 1024
H = 16
DH = 64
BS = 4
NSEL = 4
WIN = 32
NB = N // BS
NW = N // WIN
NCH = 8
CH = N // NCH
SCALE = DH ** -0.5
NEG = -1e30


def _pre_body(lam_ref, x_ref, x0_ref, wq_ref, wk_ref, wv_ref, wg_ref,
              x1_ref, q_ref, k_ref, v_ref, g_ref):
    l0 = lam_ref[0]
    l1 = lam_ref[1]
    x1 = l0 * x_ref[...] + l1 * x0_ref[...]
    x1_ref[...] = x1
    ms = jnp.mean(x1 * x1, axis=-1, keepdims=True)
    xn = x1 * lax.rsqrt(ms + 1e-6)
    q_ref[...] = jnp.dot(xn, wq_ref[...], preferred_element_type=jnp.float32)
    k_ref[...] = jnp.dot(xn, wk_ref[...], preferred_element_type=jnp.float32)
    v_ref[...] = jnp.dot(xn, wv_ref[...], preferred_element_type=jnp.float32)
    g_ref[...] = jax.nn.sigmoid(
        jnp.dot(xn, wg_ref[...], preferred_element_type=jnp.float32))


def _pre(x, x0, lam, Wq, Wk, Wv, Wg):
    row = lambda c: (c, 0)
    full = lambda c: (0, 0)
    return pl.pallas_call(
        _pre_body,
        grid=(NCH,),
        in_specs=[
            pl.BlockSpec(memory_space=pltpu.SMEM),
            pl.BlockSpec((CH, DIM), row),
            pl.BlockSpec((CH, DIM), row),
            pl.BlockSpec((DIM, H * DH), full),
            pl.BlockSpec((DIM, H * DH), full),
            pl.BlockSpec((DIM, H * DH), full),
            pl.BlockSpec((DIM, H * 3), full),
        ],
        out_specs=[
            pl.BlockSpec((CH, DIM), row),
            pl.BlockSpec((CH, H * DH), row),
            pl.BlockSpec((CH, H * DH), row),
            pl.BlockSpec((CH, H * DH), row),
            pl.BlockSpec((CH, H * 3), row),
        ],
        out_shape=[
            jax.ShapeDtypeStruct((N, DIM), jnp.float32),
            jax.ShapeDtypeStruct((N, H * DH), jnp.float32),
            jax.ShapeDtypeStruct((N, H * DH), jnp.float32),
            jax.ShapeDtypeStruct((N, H * DH), jnp.float32),
            jax.ShapeDtypeStruct((N, H * 3), jnp.float32),
        ],
    )(lam, x, x0, Wq, Wk, Wv, Wg)


def _cmp_body(q_ref, kb_ref, vb_ref, pk_ref, pv_ref, wck_ref, wcv_ref,
              mk_ref, mv_ref, oc_ref, rows_ref, kpos_ref):
    h = pl.program_id(0)
    q = q_ref[0]
    kb = kb_ref[0] + pk_ref[...]
    vb = vb_ref[0] + pv_ref[...]
    ck = jnp.dot(kb, wck_ref[...], preferred_element_type=jnp.float32)
    cv = jnp.dot(vb, wcv_ref[...], preferred_element_type=jnp.float32)
    mk = mk_ref[0]
    mv = mv_ref[0]
    sim = lax.dot_general(q, ck, (((1,), (1,)), ((), ())),
                          preferred_element_type=jnp.float32) * SCALE
    sim_m = jnp.sum(q * mk, axis=-1, keepdims=True) * SCALE
    t = lax.broadcasted_iota(jnp.int32, (N, NB), 0)
    j = lax.broadcasted_iota(jnp.int32, (N, NB), 1)
    cmask = t >= (j + 1) * BS - 1
    sim = jnp.where(cmask, sim, NEG)
    m = jnp.maximum(jnp.max(sim, axis=-1, keepdims=True), sim_m)
    e = jnp.exp(sim - m)
    e = jnp.where(cmask, e, 0.0)
    em = jnp.exp(sim_m - m)
    denom = jnp.sum(e, axis=-1, keepdims=True) + em
    attn = e / denom
    oc_ref[0] = (jnp.dot(attn, cv, preferred_element_type=jnp.float32)
                 + (em / denom) * mv)
    own = j == t // BS
    cur = jnp.where(own, 1e9, jnp.where(cmask, attn, -1.0))
    sels = []
    for _ in range(NSEL):
        mval = jnp.max(cur, axis=-1, keepdims=True)
        sel = jnp.min(jnp.where(cur == mval, j, NB), axis=-1, keepdims=True)
        sels.append(sel)
        cur = jnp.where(j == sel, -jnp.inf, cur)
    rows_ref[0] = jnp.concatenate(sels, axis=1) + h * NB
    kp = []
    for s in range(NSEL):
        for p in range(BS):
            kp.append(sels[s] * BS + p)
    kpos_ref[0] = jnp.concatenate(kp, axis=1)


def _cmp(q_t, kb4, vb4, pkf, pvf, Wck, Wcv, mem_k, mem_v):
    headN = lambda h: (h, 0, 0)
    full = lambda h: (0, 0)
    return pl.pallas_call(
        _cmp_body,
        grid=(H,),
        in_specs=[
            pl.BlockSpec((1, N, DH), headN),
            pl.BlockSpec((1, NB, BS * DH), headN),
            pl.BlockSpec((1, NB, BS * DH), headN),
            pl.BlockSpec((1, BS * DH), full),
            pl.BlockSpec((1, BS * DH), full),
            pl.BlockSpec((BS * DH, DH), full),
            pl.BlockSpec((BS * DH, DH), full),
            pl.BlockSpec((1, 1, DH), headN),
            pl.BlockSpec((1, 1, DH), headN),
        ],
        out_specs=[
            pl.BlockSpec((1, N, DH), headN),
            pl.BlockSpec((1, N, NSEL), headN),
            pl.BlockSpec((1, N, NSEL * BS), headN),
        ],
        out_shape=[
            jax.ShapeDtypeStruct((H, N, DH), jnp.float32),
            jax.ShapeDtypeStruct((H, N, NSEL), jnp.int32),
            jax.ShapeDtypeStruct((H, N, NSEL * BS), jnp.int32),
        ],
    )(q_t, kb4, vb4, pkf, pvf, Wck, Wcv, mem_k, mem_v)


GB = H * N * NSEL
ROWW = BS * DH // 2
SC_CHUNK = 128


def _sc_gather(ktab, vtab, rows):
    info = plsc.get_sparse_core_info()
    nw = info.num_cores * info.num_subcores
    bpw = GB // nw
    nchunks = bpw // SC_CHUNK
    mesh = plsc.VectorSubcoreMesh(core_axis_name="c", subcore_axis_name="s")

    @functools.partial(
        pl.kernel, mesh=mesh,
        out_type=[jax.ShapeDtypeStruct((GB, ROWW), jnp.float32),
                  jax.ShapeDtypeStruct((GB, ROWW), jnp.float32)],
        scratch_types=[pltpu.VMEM((bpw,), jnp.int32),
                       pltpu.VMEM((SC_CHUNK, ROWW), jnp.float32),
                       pltpu.VMEM((SC_CHUNK, ROWW), jnp.float32),
                       pltpu.VMEM((SC_CHUNK, ROWW), jnp.float32),
                       pltpu.VMEM((SC_CHUNK, ROWW), jnp.float32),
                       pltpu.SemaphoreType.DMA,
                       pltpu.SemaphoreType.DMA],
    )
    def body(ktab_hbm, vtab_hbm, idx_hbm, gk_hbm, gv_hbm,
             idx_v, kbuf0, vbuf0, kbuf1, vbuf1, sem0, sem1):
        wid = lax.axis_index("s") * info.num_cores + lax.axis_index("c")
        base = wid * bpw
        pltpu.sync_copy(idx_hbm.at[pl.ds(base, bpw)], idx_v)
        def step(c, carry):
            off = c * SC_CHUNK
            ck = pltpu.async_copy(
                ktab_hbm.at[idx_v.at[pl.ds(off, SC_CHUNK)]], kbuf0, sem0)
            cv = pltpu.async_copy(
                vtab_hbm.at[idx_v.at[pl.ds(off, SC_CHUNK)]], vbuf0, sem0)
            ck.wait()
            cv.wait()
            pltpu.sync_copy(kbuf0, gk_hbm.at[pl.ds(base + off, SC_CHUNK)])
            pltpu.sync_copy(vbuf0, gv_hbm.at[pl.ds(base + off, SC_CHUNK)])
            return carry

        lax.fori_loop(0, nchunks, step, 0)
        del kbuf1, vbuf1, sem1

    return body(ktab, vtab, rows)


def _fin_body(q_ref, k_ref, v_ref, gk_ref, gv_ref, kpos_ref, oc_ref, g_ref,
              out_ref):
    c = pl.program_id(1)
    q = q_ref[0]
    gk = gk_ref[0].astype(jnp.float32)
    gv = gv_ref[0].astype(jnp.float32)
    kpos = kpos_ref[0]
    qpos = c * CH + lax.broadcasted_iota(jnp.int32, (CH, NSEL * BS), 0)
    simf = jnp.sum(gk * q[:, None, :], axis=-1) * SCALE
    simf = jnp.where(kpos <= qpos, simf, NEG)
    mf = jnp.max(simf, axis=-1, keepdims=True)
    ef = jnp.exp(simf - mf)
    pf = ef / jnp.sum(ef, axis=-1, keepdims=True)
    out_f = jnp.sum(pf[:, :, None] * gv, axis=1)
    outs_w = []
    for w in range(CH // WIN):
        W = c * (CH // WIN) + w
        base = jnp.maximum(W * WIN - WIN, 0)
        qb = q[w * WIN:(w + 1) * WIN]
        kb = k_ref[0, pl.ds(base, 2 * WIN), :]
        vb = v_ref[0, pl.ds(base, 2 * WIN), :]
        sw = lax.dot_general(qb, kb, (((1,), (1,)), ((), ())),
                             preferred_element_type=jnp.float32) * SCALE
        sg = W * WIN + lax.broadcasted_iota(jnp.int32, (WIN, 2 * WIN), 0)
        tg = base + lax.broadcasted_iota(jnp.int32, (WIN, 2 * WIN), 1)
        wm = (tg >= sg - (WIN - 1)) & (tg <= sg)
        sw = jnp.where(wm, sw, NEG)
        mw = jnp.max(sw, axis=-1, keepdims=True)
        ew = jnp.exp(sw - mw)
        pw = ew / jnp.sum(ew, axis=-1, keepdims=True)
        outs_w.append(jnp.dot(pw, vb, preferred_element_type=jnp.float32))
    out_w = jnp.concatenate(outs_w, axis=0)
    g = g_ref[0]
    out_ref[0] = (g[:, 0:1] * oc_ref[0] + g[:, 1:2] * out_f
                  + g[:, 2:3] * out_w)


def _fin(q_t, k_t, v_t, gk4, gv4, kpos, out_c, g_t):
    hc = lambda h, c: (h, c, 0)
    hfull = lambda h, c: (h, 0, 0)
    hc4 = lambda h, c: (h, c, 0, 0)
    return pl.pallas_call(
        _fin_body,
        grid=(H, NCH),
        in_specs=[
            pl.BlockSpec((1, CH, DH), hc),
            pl.BlockSpec((1, N, DH), hfull),
            pl.BlockSpec((1, N, DH), hfull),
            pl.BlockSpec((1, CH, NSEL * BS, DH), hc4),
            pl.BlockSpec((1, CH, NSEL * BS, DH), hc4),
            pl.BlockSpec((1, CH, NSEL * BS), hc),
            pl.BlockSpec((1, CH, DH), hc),
            pl.BlockSpec((1, CH, 3), hc),
        ],
        out_specs=pl.BlockSpec((1, CH, DH), hc),
        out_shape=jax.ShapeDtypeStruct((H, N, DH), jnp.float32),
    )(q_t, k_t, v_t, gk4, gv4, kpos, out_c, g_t)


def _mlp_body(x1_ref, att_ref, wo_ref, w1_ref, w2_ref, y_ref):
    att16 = att_ref[...].astype(jnp.bfloat16)
    x2 = x1_ref[...] + jnp.dot(att16, wo_ref[...],
                               preferred_element_type=jnp.float32)
    ms = jnp.mean(x2 * x2, axis=-1, keepdims=True)
    xn = x2 * lax.rsqrt(ms + 1e-6)
    hh = jnp.maximum(jnp.dot(xn.astype(jnp.bfloat16), w1_ref[...],
                             preferred_element_type=jnp.float32), 0.0)
    hh = hh * hh
    y_ref[...] = x2 + jnp.dot(hh.astype(jnp.bfloat16), w2_ref[...],
                              preferred_element_type=jnp.float32)


def _mlp(x1, att, Wo, W1, W2):
    row = lambda c: (c, 0)
    full = lambda c: (0, 0)
    return pl.pallas_call(
        _mlp_body,
        grid=(NCH,),
        in_specs=[
            pl.BlockSpec((CH, DIM), row),
            pl.BlockSpec((CH, DIM), row),
            pl.BlockSpec((DIM, DIM), full),
            pl.BlockSpec((DIM, 4 * DIM), full),
            pl.BlockSpec((4 * DIM, DIM), full),
        ],
        out_specs=pl.BlockSpec((CH, DIM), row),
        out_shape=jax.ShapeDtypeStruct((N, DIM), jnp.float32),
    )(x1, att, Wo, W1, W2)


def kernel(x, ve, x0, block_mask, lambdas, Wq, Wk, Wv, Wo, Wg, pos_k, pos_v,
           Wck, Wcv, mem_k, mem_v, W1, W2):
    xr = x.reshape(N, DIM)
    x0r = x0.reshape(N, DIM)
    x1, q, k, v, g48 = _pre(xr, x0r, lambdas, Wq, Wk, Wv, Wg)
    q_t = q.reshape(N, H, DH).transpose(1, 0, 2)
    k_t = k.reshape(N, H, DH).transpose(1, 0, 2)
    v_t = v.reshape(N, H, DH).transpose(1, 0, 2)
    g_t = g48.reshape(N, H, 3).transpose(1, 0, 2)
    kb4 = k_t.reshape(H, NB, BS * DH)
    vb4 = v_t.reshape(H, NB, BS * DH)
    pkf = pos_k.reshape(1, BS * DH)
    pvf = pos_v.reshape(1, BS * DH)
    out_c, rows, kpos = _cmp(q_t, kb4, vb4, pkf, pvf, Wck, Wcv, mem_k, mem_v)
    ktab = lax.bitcast_convert_type(
        k_t.astype(jnp.bfloat16).reshape(H * NB, ROWW, 2), jnp.float32)
    vtab = lax.bitcast_convert_type(
        v_t.astype(jnp.bfloat16).reshape(H * NB, ROWW, 2), jnp.float32)
    rflat = rows.reshape(-1)
    gk, gv = _sc_gather(ktab, vtab, rflat)
    gk4 = lax.bitcast_convert_type(gk, jnp.bfloat16).reshape(
        H, N, NSEL * BS, DH)
    gv4 = lax.bitcast_convert_type(gv, jnp.bfloat16).reshape(
        H, N, NSEL * BS, DH)
    att = _fin(q_t, k_t, v_t, gk4, gv4, kpos, out_c, g_t)
    att2 = att.transpose(1, 0, 2).reshape(N, H * DH)
    y = _mlp(x1, att2, Wo.astype(jnp.bfloat16), W1.astype(jnp.bfloat16),
             W2.astype(jnp.bfloat16))
    return y.reshape(1, N, DIM)


# speedup vs baseline: 19.5824x; 19.5824x over previous
import functools
import jax
import jax.numpy as jnp
from jax import lax
from jax.experimental import pallas as pl
from jax.experimental.pallas import tpu as pltpu
from jax.experimental.pallas import tpu_sc as plsc

N = 2048
DIM = 1024
H = 16
DH = 64
BS = 4
NSEL = 4
WIN = 32
NB = N // BS
NW = N // WIN
NCH = 8
CH = N // NCH
SCALE = DH ** -0.5
NEG = -1e30


def _pre_body(lam_ref, x_ref, x0_ref, wq_ref, wk_ref, wv_ref, wg_ref,
              x1_ref, q_ref, k_ref, v_ref, g_ref):
    l0 = lam_ref[0]
    l1 = lam_ref[1]
    x1 = l0 * x_ref[...] + l1 * x0_ref[...]
    x1_ref[...] = x1
    ms = jnp.mean(x1 * x1, axis=-1, keepdims=True)
    xn = x1 * lax.rsqrt(ms + 1e-6)
    q_ref[...] = jnp.dot(xn, wq_ref[...], preferred_element_type=jnp.float32)
    k_ref[...] = jnp.dot(xn, wk_ref[...], preferred_element_type=jnp.float32)
    v_ref[...] = jnp.dot(xn, wv_ref[...], preferred_element_type=jnp.float32)
    g_ref[...] = jax.nn.sigmoid(
        jnp.dot(xn, wg_ref[...], preferred_element_type=jnp.float32))


def _pre(x, x0, lam, Wq, Wk, Wv, Wg):
    row = lambda c: (c, 0)
    full = lambda c: (0, 0)
    return pl.pallas_call(
        _pre_body,
        grid=(NCH,),
        in_specs=[
            pl.BlockSpec(memory_space=pltpu.SMEM),
            pl.BlockSpec((CH, DIM), row),
            pl.BlockSpec((CH, DIM), row),
            pl.BlockSpec((DIM, H * DH), full),
            pl.BlockSpec((DIM, H * DH), full),
            pl.BlockSpec((DIM, H * DH), full),
            pl.BlockSpec((DIM, H * 3), full),
        ],
        out_specs=[
            pl.BlockSpec((CH, DIM), row),
            pl.BlockSpec((CH, H * DH), row),
            pl.BlockSpec((CH, H * DH), row),
            pl.BlockSpec((CH, H * DH), row),
            pl.BlockSpec((CH, H * 3), row),
        ],
        out_shape=[
            jax.ShapeDtypeStruct((N, DIM), jnp.float32),
            jax.ShapeDtypeStruct((N, H * DH), jnp.float32),
            jax.ShapeDtypeStruct((N, H * DH), jnp.float32),
            jax.ShapeDtypeStruct((N, H * DH), jnp.float32),
            jax.ShapeDtypeStruct((N, H * 3), jnp.float32),
        ],
    )(lam, x, x0, Wq, Wk, Wv, Wg)


def _cmp_body(q_ref, kb_ref, vb_ref, pk_ref, pv_ref, wck_ref, wcv_ref,
              mk_ref, mv_ref, oc_ref, rows_ref, kpos_ref):
    h = pl.program_id(0)
    q = q_ref[0]
    kb = kb_ref[0] + pk_ref[...]
    vb = vb_ref[0] + pv_ref[...]
    ck = jnp.dot(kb, wck_ref[...], preferred_element_type=jnp.float32)
    cv = jnp.dot(vb, wcv_ref[...], preferred_element_type=jnp.float32)
    mk = mk_ref[0]
    mv = mv_ref[0]
    sim = lax.dot_general(q, ck, (((1,), (1,)), ((), ())),
                          preferred_element_type=jnp.float32) * SCALE
    sim_m = jnp.sum(q * mk, axis=-1, keepdims=True) * SCALE
    t = lax.broadcasted_iota(jnp.int32, (N, NB), 0)
    j = lax.broadcasted_iota(jnp.int32, (N, NB), 1)
    cmask = t >= (j + 1) * BS - 1
    sim = jnp.where(cmask, sim, NEG)
    m = jnp.maximum(jnp.max(sim, axis=-1, keepdims=True), sim_m)
    e = jnp.exp(sim - m)
    e = jnp.where(cmask, e, 0.0)
    em = jnp.exp(sim_m - m)
    denom = jnp.sum(e, axis=-1, keepdims=True) + em
    attn = e / denom
    oc_ref[0] = (jnp.dot(attn, cv, preferred_element_type=jnp.float32)
                 + (em / denom) * mv)
    own = j == t // BS
    cur = jnp.where(own, 1e9, jnp.where(cmask, attn, -1.0))
    sels = []
    for _ in range(NSEL):
        mval = jnp.max(cur, axis=-1, keepdims=True)
        sel = jnp.min(jnp.where(cur == mval, j, NB), axis=-1, keepdims=True)
        sels.append(sel)
        cur = jnp.where(j == sel, -jnp.inf, cur)
    rows_ref[0] = jnp.concatenate(sels, axis=1) + h * NB
    kp = []
    for s in range(NSEL):
        for p in range(BS):
            kp.append(sels[s] * BS + p)
    kpos_ref[0] = jnp.concatenate(kp, axis=1)


def _cmp(q_t, kb4, vb4, pkf, pvf, Wck, Wcv, mem_k, mem_v):
    headN = lambda h: (h, 0, 0)
    full = lambda h: (0, 0)
    return pl.pallas_call(
        _cmp_body,
        grid=(H,),
        in_specs=[
            pl.BlockSpec((1, N, DH), headN),
            pl.BlockSpec((1, NB, BS * DH), headN),
            pl.BlockSpec((1, NB, BS * DH), headN),
            pl.BlockSpec((1, BS * DH), full),
            pl.BlockSpec((1, BS * DH), full),
            pl.BlockSpec((BS * DH, DH), full),
            pl.BlockSpec((BS * DH, DH), full),
            pl.BlockSpec((1, 1, DH), headN),
            pl.BlockSpec((1, 1, DH), headN),
        ],
        out_specs=[
            pl.BlockSpec((1, N, DH), headN),
            pl.BlockSpec((1, N, NSEL), headN),
            pl.BlockSpec((1, N, NSEL * BS), headN),
        ],
        out_shape=[
            jax.ShapeDtypeStruct((H, N, DH), jnp.float32),
            jax.ShapeDtypeStruct((H, N, NSEL), jnp.int32),
            jax.ShapeDtypeStruct((H, N, NSEL * BS), jnp.int32),
        ],
    )(q_t, kb4, vb4, pkf, pvf, Wck, Wcv, mem_k, mem_v)


GB = H * N * NSEL
ROWW = BS * DH
SC_CHUNK = 64


def _sc_gather(ktab, vtab, rows):
    info = plsc.get_sparse_core_info()
    nw = info.num_cores * info.num_subcores
    bpw = GB // nw
    nchunks = bpw // SC_CHUNK
    mesh = plsc.VectorSubcoreMesh(core_axis_name="c", subcore_axis_name="s")

    @functools.partial(
        pl.kernel, mesh=mesh,
        out_type=[jax.ShapeDtypeStruct((GB, ROWW), jnp.float32),
                  jax.ShapeDtypeStruct((GB, ROWW), jnp.float32)],
        scratch_types=[pltpu.VMEM((bpw,), jnp.int32),
                       pltpu.VMEM((SC_CHUNK, ROWW), jnp.float32),
                       pltpu.VMEM((SC_CHUNK, ROWW), jnp.float32),
                       pltpu.VMEM((SC_CHUNK, ROWW), jnp.float32),
                       pltpu.VMEM((SC_CHUNK, ROWW), jnp.float32),
                       pltpu.SemaphoreType.DMA,
                       pltpu.SemaphoreType.DMA],
    )
    def body(ktab_hbm, vtab_hbm, idx_hbm, gk_hbm, gv_hbm,
             idx_v, kbuf0, vbuf0, kbuf1, vbuf1, sem0, sem1):
        wid = lax.axis_index("s") * info.num_cores + lax.axis_index("c")
        base = wid * bpw
        pltpu.sync_copy(idx_hbm.at[pl.ds(base, bpw)], idx_v)
        bufs = [(kbuf0, vbuf0, sem0), (kbuf1, vbuf1, sem1)]
        handles = {}

        def issue(c):
            kb, vb, sm = bufs[c % 2]
            idx_slice = idx_v.at[pl.ds(c * SC_CHUNK, SC_CHUNK)]
            handles[c] = (pltpu.async_copy(ktab_hbm.at[idx_slice], kb, sm),
                          pltpu.async_copy(vtab_hbm.at[idx_slice], vb, sm))

        issue(0)
        for c in range(nchunks):
            if c + 1 < nchunks:
                issue(c + 1)
            kb, vb, _ = bufs[c % 2]
            hk, hv = handles.pop(c)
            hk.wait()
            hv.wait()
            pltpu.sync_copy(kb, gk_hbm.at[pl.ds(base + c * SC_CHUNK, SC_CHUNK)])
            pltpu.sync_copy(vb, gv_hbm.at[pl.ds(base + c * SC_CHUNK, SC_CHUNK)])

    return body(ktab, vtab, rows)


def _fin_body(q_ref, k_ref, v_ref, gk_ref, gv_ref, kpos_ref, oc_ref, g_ref,
              out_ref):
    c = pl.program_id(1)
    q = q_ref[0]
    gk = gk_ref[0]
    gv = gv_ref[0]
    kpos = kpos_ref[0]
    qpos = c * CH + lax.broadcasted_iota(jnp.int32, (CH, NSEL * BS), 0)
    simf = jnp.sum(gk * q[:, None, :], axis=-1) * SCALE
    simf = jnp.where(kpos <= qpos, simf, NEG)
    mf = jnp.max(simf, axis=-1, keepdims=True)
    ef = jnp.exp(simf - mf)
    pf = ef / jnp.sum(ef, axis=-1, keepdims=True)
    out_f = jnp.sum(pf[:, :, None] * gv, axis=1)
    outs_w = []
    for w in range(CH // WIN):
        W = c * (CH // WIN) + w
        base = jnp.maximum(W * WIN - WIN, 0)
        qb = q[w * WIN:(w + 1) * WIN]
        kb = k_ref[0, pl.ds(base, 2 * WIN), :]
        vb = v_ref[0, pl.ds(base, 2 * WIN), :]
        sw = lax.dot_general(qb, kb, (((1,), (1,)), ((), ())),
                             preferred_element_type=jnp.float32) * SCALE
        sg = W * WIN + lax.broadcasted_iota(jnp.int32, (WIN, 2 * WIN), 0)
        tg = base + lax.broadcasted_iota(jnp.int32, (WIN, 2 * WIN), 1)
        wm = (tg >= sg - (WIN - 1)) & (tg <= sg)
        sw = jnp.where(wm, sw, NEG)
        mw = jnp.max(sw, axis=-1, keepdims=True)
        ew = jnp.exp(sw - mw)
        pw = ew / jnp.sum(ew, axis=-1, keepdims=True)
        outs_w.append(jnp.dot(pw, vb, preferred_element_type=jnp.float32))
    out_w = jnp.concatenate(outs_w, axis=0)
    g = g_ref[0]
    out_ref[0] = (g[:, 0:1] * oc_ref[0] + g[:, 1:2] * out_f
                  + g[:, 2:3] * out_w)


def _fin(q_t, k_t, v_t, gk4, gv4, kpos, out_c, g_t):
    hc = lambda h, c: (h, c, 0)
    hfull = lambda h, c: (h, 0, 0)
    hc4 = lambda h, c: (h, c, 0, 0)
    return pl.pallas_call(
        _fin_body,
        grid=(H, NCH),
        in_specs=[
            pl.BlockSpec((1, CH, DH), hc),
            pl.BlockSpec((1, N, DH), hfull),
            pl.BlockSpec((1, N, DH), hfull),
            pl.BlockSpec((1, CH, NSEL * BS, DH), hc4),
            pl.BlockSpec((1, CH, NSEL * BS, DH), hc4),
            pl.BlockSpec((1, CH, NSEL * BS), hc),
            pl.BlockSpec((1, CH, DH), hc),
            pl.BlockSpec((1, CH, 3), hc),
        ],
        out_specs=pl.BlockSpec((1, CH, DH), hc),
        out_shape=jax.ShapeDtypeStruct((H, N, DH), jnp.float32),
    )(q_t, k_t, v_t, gk4, gv4, kpos, out_c, g_t)


def _mlp_body(x1_ref, att_ref, wo_ref, w1_ref, w2_ref, y_ref):
    att16 = att_ref[...].astype(jnp.bfloat16)
    x2 = x1_ref[...] + jnp.dot(att16, wo_ref[...],
                               preferred_element_type=jnp.float32)
    ms = jnp.mean(x2 * x2, axis=-1, keepdims=True)
    xn = x2 * lax.rsqrt(ms + 1e-6)
    hh = jnp.maximum(jnp.dot(xn.astype(jnp.bfloat16), w1_ref[...],
                             preferred_element_type=jnp.float32), 0.0)
    hh = hh * hh
    y_ref[...] = x2 + jnp.dot(hh.astype(jnp.bfloat16), w2_ref[...],
                              preferred_element_type=jnp.float32)


def _mlp(x1, att, Wo, W1, W2):
    row = lambda c: (c, 0)
    full = lambda c: (0, 0)
    return pl.pallas_call(
        _mlp_body,
        grid=(NCH,),
        in_specs=[
            pl.BlockSpec((CH, DIM), row),
            pl.BlockSpec((CH, DIM), row),
            pl.BlockSpec((DIM, DIM), full),
            pl.BlockSpec((DIM, 4 * DIM), full),
            pl.BlockSpec((4 * DIM, DIM), full),
        ],
        out_specs=pl.BlockSpec((CH, DIM), row),
        out_shape=jax.ShapeDtypeStruct((N, DIM), jnp.float32),
    )(x1, att, Wo, W1, W2)


def kernel(x, ve, x0, block_mask, lambdas, Wq, Wk, Wv, Wo, Wg, pos_k, pos_v,
           Wck, Wcv, mem_k, mem_v, W1, W2):
    xr = x.reshape(N, DIM)
    x0r = x0.reshape(N, DIM)
    x1, q, k, v, g48 = _pre(xr, x0r, lambdas, Wq, Wk, Wv, Wg)
    q_t = q.reshape(N, H, DH).transpose(1, 0, 2)
    k_t = k.reshape(N, H, DH).transpose(1, 0, 2)
    v_t = v.reshape(N, H, DH).transpose(1, 0, 2)
    g_t = g48.reshape(N, H, 3).transpose(1, 0, 2)
    kb4 = k_t.reshape(H, NB, BS * DH)
    vb4 = v_t.reshape(H, NB, BS * DH)
    pkf = pos_k.reshape(1, BS * DH)
    pvf = pos_v.reshape(1, BS * DH)
    out_c, rows, kpos = _cmp(q_t, kb4, vb4, pkf, pvf, Wck, Wcv, mem_k, mem_v)
    ktab = k_t.reshape(H * NB, BS * DH)
    vtab = v_t.reshape(H * NB, BS * DH)
    rflat = rows.reshape(-1)
    gk, gv = _sc_gather(ktab, vtab, rflat)
    gk4 = gk.reshape(H, N, NSEL * BS, DH)
    gv4 = gv.reshape(H, N, NSEL * BS, DH)
    att = _fin(q_t, k_t, v_t, gk4, gv4, kpos, out_c, g_t)
    att2 = att.transpose(1, 0, 2).reshape(N, H * DH)
    y = _mlp(x1, att2, Wo.astype(jnp.bfloat16), W1.astype(jnp.bfloat16),
             W2.astype(jnp.bfloat16))
    return y.reshape(1, N, DIM)


# speedup vs baseline: 20.9289x; 1.0688x over previous
import functools
import jax
import jax.numpy as jnp
from jax import lax
from jax.experimental import pallas as pl
from jax.experimental.pallas import tpu as pltpu
from jax.experimental.pallas import tpu_sc as plsc

N = 2048
DIM = 1024
H = 16
DH = 64
BS = 4
NSEL = 4
WIN = 32
NB = N // BS
NW = N // WIN
NCH = 8
CH = N // NCH
SCALE = DH ** -0.5
NEG = -1e30


def _pre_body(lam_ref, x_ref, x0_ref, wq_ref, wk_ref, wv_ref, wg_ref,
              x1_ref, q_ref, k_ref, v_ref, g_ref):
    l0 = lam_ref[0]
    l1 = lam_ref[1]
    x1 = l0 * x_ref[...] + l1 * x0_ref[...]
    x1_ref[...] = x1
    ms = jnp.mean(x1 * x1, axis=-1, keepdims=True)
    xn = x1 * lax.rsqrt(ms + 1e-6)
    q_ref[...] = jnp.dot(xn, wq_ref[...], preferred_element_type=jnp.float32)
    k_ref[...] = jnp.dot(xn, wk_ref[...], preferred_element_type=jnp.float32)
    v_ref[...] = jnp.dot(xn, wv_ref[...], preferred_element_type=jnp.float32)
    g_ref[...] = jax.nn.sigmoid(
        jnp.dot(xn, wg_ref[...], preferred_element_type=jnp.float32))


def _pre(x, x0, lam, Wq, Wk, Wv, Wg):
    row = lambda c: (c, 0)
    full = lambda c: (0, 0)
    return pl.pallas_call(
        _pre_body,
        grid=(NCH,),
        in_specs=[
            pl.BlockSpec(memory_space=pltpu.SMEM),
            pl.BlockSpec((CH, DIM), row),
            pl.BlockSpec((CH, DIM), row),
            pl.BlockSpec((DIM, H * DH), full),
            pl.BlockSpec((DIM, H * DH), full),
            pl.BlockSpec((DIM, H * DH), full),
            pl.BlockSpec((DIM, H * 3), full),
        ],
        out_specs=[
            pl.BlockSpec((CH, DIM), row),
            pl.BlockSpec((CH, H * DH), row),
            pl.BlockSpec((CH, H * DH), row),
            pl.BlockSpec((CH, H * DH), row),
            pl.BlockSpec((CH, H * 3), row),
        ],
        out_shape=[
            jax.ShapeDtypeStruct((N, DIM), jnp.float32),
            jax.ShapeDtypeStruct((N, H * DH), jnp.float32),
            jax.ShapeDtypeStruct((N, H * DH), jnp.float32),
            jax.ShapeDtypeStruct((N, H * DH), jnp.float32),
            jax.ShapeDtypeStruct((N, H * 3), jnp.float32),
        ],
    )(lam, x, x0, Wq, Wk, Wv, Wg)


def _cmp_body(q_ref, kb_ref, vb_ref, pk_ref, pv_ref, wck_ref, wcv_ref,
              mk_ref, mv_ref, oc_ref, rows_ref, kpos_ref):
    h = pl.program_id(0)
    q = q_ref[0]
    kb = kb_ref[0] + pk_ref[...]
    vb = vb_ref[0] + pv_ref[...]
    ck = jnp.dot(kb, wck_ref[...], preferred_element_type=jnp.float32)
    cv = jnp.dot(vb, wcv_ref[...], preferred_element_type=jnp.float32)
    mk = mk_ref[0]
    mv = mv_ref[0]
    sim = lax.dot_general(q, ck, (((1,), (1,)), ((), ())),
                          preferred_element_type=jnp.float32) * SCALE
    sim_m = jnp.sum(q * mk, axis=-1, keepdims=True) * SCALE
    t = lax.broadcasted_iota(jnp.int32, (N, NB), 0)
    j = lax.broadcasted_iota(jnp.int32, (N, NB), 1)
    cmask = t >= (j + 1) * BS - 1
    sim = jnp.where(cmask, sim, NEG)
    m = jnp.maximum(jnp.max(sim, axis=-1, keepdims=True), sim_m)
    e = jnp.exp(sim - m)
    e = jnp.where(cmask, e, 0.0)
    em = jnp.exp(sim_m - m)
    denom = jnp.sum(e, axis=-1, keepdims=True) + em
    attn = e / denom
    oc_ref[0] = (jnp.dot(attn, cv, preferred_element_type=jnp.float32)
                 + (em / denom) * mv)
    own = j == t // BS
    cur = jnp.where(own, 1e9, jnp.where(cmask, attn, -1.0))
    sels = []
    for _ in range(NSEL):
        mval = jnp.max(cur, axis=-1, keepdims=True)
        sel = jnp.min(jnp.where(cur == mval, j, NB), axis=-1, keepdims=True)
        sels.append(sel)
        cur = jnp.where(j == sel, -jnp.inf, cur)
    rows_ref[0] = jnp.concatenate(sels, axis=1) + h * NB
    kp = []
    for s in range(NSEL):
        for p in range(BS):
            kp.append(sels[s] * BS + p)
    kpos_ref[0] = jnp.concatenate(kp, axis=1)


def _cmp(q_t, kb4, vb4, pkf, pvf, Wck, Wcv, mem_k, mem_v):
    headN = lambda h: (h, 0, 0)
    full = lambda h: (0, 0)
    return pl.pallas_call(
        _cmp_body,
        grid=(H,),
        in_specs=[
            pl.BlockSpec((1, N, DH), headN),
            pl.BlockSpec((1, NB, BS * DH), headN),
            pl.BlockSpec((1, NB, BS * DH), headN),
            pl.BlockSpec((1, BS * DH), full),
            pl.BlockSpec((1, BS * DH), full),
            pl.BlockSpec((BS * DH, DH), full),
            pl.BlockSpec((BS * DH, DH), full),
            pl.BlockSpec((1, 1, DH), headN),
            pl.BlockSpec((1, 1, DH), headN),
        ],
        out_specs=[
            pl.BlockSpec((1, N, DH), headN),
            pl.BlockSpec((1, N, NSEL), headN),
            pl.BlockSpec((1, N, NSEL * BS), headN),
        ],
        out_shape=[
            jax.ShapeDtypeStruct((H, N, DH), jnp.float32),
            jax.ShapeDtypeStruct((H, N, NSEL), jnp.int32),
            jax.ShapeDtypeStruct((H, N, NSEL * BS), jnp.int32),
        ],
    )(q_t, kb4, vb4, pkf, pvf, Wck, Wcv, mem_k, mem_v)


GB = H * N * NSEL
ROWW = BS * DH
SC_CHUNK = 64


def _sc_gather(ktab, vtab, rows):
    info = plsc.get_sparse_core_info()
    nw = info.num_cores * info.num_subcores
    bpw = GB // nw
    nchunks = bpw // SC_CHUNK
    mesh = plsc.VectorSubcoreMesh(core_axis_name="c", subcore_axis_name="s")

    @functools.partial(
        pl.kernel, mesh=mesh,
        out_type=[jax.ShapeDtypeStruct((GB, ROWW), jnp.float32),
                  jax.ShapeDtypeStruct((GB, ROWW), jnp.float32)],
        scratch_types=[pltpu.VMEM((bpw,), jnp.int32),
                       pltpu.VMEM((SC_CHUNK, ROWW), jnp.float32),
                       pltpu.VMEM((SC_CHUNK, ROWW), jnp.float32),
                       pltpu.VMEM((SC_CHUNK, ROWW), jnp.float32),
                       pltpu.VMEM((SC_CHUNK, ROWW), jnp.float32),
                       pltpu.SemaphoreType.DMA,
                       pltpu.SemaphoreType.DMA],
    )
    def body(ktab_hbm, vtab_hbm, idx_hbm, gk_hbm, gv_hbm,
             idx_v, kbuf0, vbuf0, kbuf1, vbuf1, sem0, sem1):
        wid = lax.axis_index("s") * info.num_cores + lax.axis_index("c")
        base = wid * bpw
        pltpu.sync_copy(idx_hbm.at[pl.ds(base, bpw)], idx_v)
        bufs = [(kbuf0, vbuf0, sem0), (kbuf1, vbuf1, sem1)]
        handles = {}

        def issue(c):
            kb, vb, sm = bufs[c % 2]
            idx_slice = idx_v.at[pl.ds(c * SC_CHUNK, SC_CHUNK)]
            handles[c] = (pltpu.async_copy(ktab_hbm.at[idx_slice], kb, sm),
                          pltpu.async_copy(vtab_hbm.at[idx_slice], vb, sm))

        issue(0)
        for c in range(nchunks):
            if c + 1 < nchunks:
                issue(c + 1)
            kb, vb, _ = bufs[c % 2]
            hk, hv = handles.pop(c)
            hk.wait()
            hv.wait()
            pltpu.sync_copy(kb, gk_hbm.at[pl.ds(base + c * SC_CHUNK, SC_CHUNK)])
            pltpu.sync_copy(vb, gv_hbm.at[pl.ds(base + c * SC_CHUNK, SC_CHUNK)])

    return body(ktab, vtab, rows)


def _fin_body(q_ref, k_ref, v_ref, gk_ref, gv_ref, kpos_ref, oc_ref, g_ref,
              out_ref):
    c = pl.program_id(1)
    q = q_ref[0]
    gk = gk_ref[0]
    gv = gv_ref[0]
    kpos = kpos_ref[0]
    qpos = c * CH + lax.broadcasted_iota(jnp.int32, (CH, NSEL * BS), 0)
    simf = jnp.sum(gk * q[:, None, :], axis=-1) * SCALE
    simf = jnp.where(kpos <= qpos, simf, NEG)
    mf = jnp.max(simf, axis=-1, keepdims=True)
    ef = jnp.exp(simf - mf)
    pf = ef / jnp.sum(ef, axis=-1, keepdims=True)
    out_f = jnp.sum(pf[:, :, None] * gv, axis=1)
    base = jnp.maximum(c * CH - WIN, 0)
    kb = k_ref[0, pl.ds(base, CH + WIN), :]
    vb = v_ref[0, pl.ds(base, CH + WIN), :]
    sw = lax.dot_general(q, kb, (((1,), (1,)), ((), ())),
                         preferred_element_type=jnp.float32) * SCALE
    sg = c * CH + lax.broadcasted_iota(jnp.int32, (CH, CH + WIN), 0)
    tg = base + lax.broadcasted_iota(jnp.int32, (CH, CH + WIN), 1)
    wm = (tg >= sg - (WIN - 1)) & (tg <= sg)
    sw = jnp.where(wm, sw, NEG)
    mw = jnp.max(sw, axis=-1, keepdims=True)
    ew = jnp.exp(sw - mw)
    ew = jnp.where(wm, ew, 0.0)
    pw = ew / jnp.sum(ew, axis=-1, keepdims=True)
    out_w = jnp.dot(pw, vb, preferred_element_type=jnp.float32)
    g = g_ref[0]
    out_ref[0] = (g[:, 0:1] * oc_ref[0] + g[:, 1:2] * out_f
                  + g[:, 2:3] * out_w)


def _fin(q_t, k_t, v_t, gk4, gv4, kpos, out_c, g_t):
    hc = lambda h, c: (h, c, 0)
    hfull = lambda h, c: (h, 0, 0)
    hc4 = lambda h, c: (h, c, 0, 0)
    return pl.pallas_call(
        _fin_body,
        grid=(H, NCH),
        in_specs=[
            pl.BlockSpec((1, CH, DH), hc),
            pl.BlockSpec((1, N, DH), hfull),
            pl.BlockSpec((1, N, DH), hfull),
            pl.BlockSpec((1, CH, NSEL * BS, DH), hc4),
            pl.BlockSpec((1, CH, NSEL * BS, DH), hc4),
            pl.BlockSpec((1, CH, NSEL * BS), hc),
            pl.BlockSpec((1, CH, DH), hc),
            pl.BlockSpec((1, CH, 3), hc),
        ],
        out_specs=pl.BlockSpec((1, CH, DH), hc),
        out_shape=jax.ShapeDtypeStruct((H, N, DH), jnp.float32),
    )(q_t, k_t, v_t, gk4, gv4, kpos, out_c, g_t)


def _mlp_body(x1_ref, att_ref, wo_ref, w1_ref, w2_ref, y_ref):
    att16 = att_ref[...].astype(jnp.bfloat16)
    x2 = x1_ref[...] + jnp.dot(att16, wo_ref[...].astype(jnp.bfloat16),
                               preferred_element_type=jnp.float32)
    ms = jnp.mean(x2 * x2, axis=-1, keepdims=True)
    xn = x2 * lax.rsqrt(ms + 1e-6)
    hh = jnp.maximum(jnp.dot(xn.astype(jnp.bfloat16),
                             w1_ref[...].astype(jnp.bfloat16),
                             preferred_element_type=jnp.float32), 0.0)
    hh = hh * hh
    y_ref[...] = x2 + jnp.dot(hh.astype(jnp.bfloat16),
                              w2_ref[...].astype(jnp.bfloat16),
                              preferred_element_type=jnp.float32)


def _mlp(x1, att, Wo, W1, W2):
    row = lambda c: (c, 0)
    full = lambda c: (0, 0)
    return pl.pallas_call(
        _mlp_body,
        grid=(NCH,),
        in_specs=[
            pl.BlockSpec((CH, DIM), row),
            pl.BlockSpec((CH, DIM), row),
            pl.BlockSpec((DIM, DIM), full),
            pl.BlockSpec((DIM, 4 * DIM), full),
            pl.BlockSpec((4 * DIM, DIM), full),
        ],
        out_specs=pl.BlockSpec((CH, DIM), row),
        out_shape=jax.ShapeDtypeStruct((N, DIM), jnp.float32),
    )(x1, att, Wo, W1, W2)


def kernel(x, ve, x0, block_mask, lambdas, Wq, Wk, Wv, Wo, Wg, pos_k, pos_v,
           Wck, Wcv, mem_k, mem_v, W1, W2):
    xr = x.reshape(N, DIM)
    x0r = x0.reshape(N, DIM)
    x1, q, k, v, g48 = _pre(xr, x0r, lambdas, Wq, Wk, Wv, Wg)
    q_t = q.reshape(N, H, DH).transpose(1, 0, 2)
    k_t = k.reshape(N, H, DH).transpose(1, 0, 2)
    v_t = v.reshape(N, H, DH).transpose(1, 0, 2)
    g_t = g48.reshape(N, H, 3).transpose(1, 0, 2)
    kb4 = k_t.reshape(H, NB, BS * DH)
    vb4 = v_t.reshape(H, NB, BS * DH)
    pkf = pos_k.reshape(1, BS * DH)
    pvf = pos_v.reshape(1, BS * DH)
    out_c, rows, kpos = _cmp(q_t, kb4, vb4, pkf, pvf, Wck, Wcv, mem_k, mem_v)
    ktab = k_t.reshape(H * NB, BS * DH)
    vtab = v_t.reshape(H * NB, BS * DH)
    rflat = rows.reshape(-1)
    gk, gv = _sc_gather(ktab, vtab, rflat)
    gk4 = gk.reshape(H, N, NSEL * BS, DH)
    gv4 = gv.reshape(H, N, NSEL * BS, DH)
    att = _fin(q_t, k_t, v_t, gk4, gv4, kpos, out_c, g_t)
    att2 = att.transpose(1, 0, 2).reshape(N, H * DH)
    y = _mlp(x1, att2, Wo, W1, W2)
    return y.reshape(1, N, DIM)


# speedup vs baseline: 21.1795x; 1.0120x over previous
import functools
import jax
import jax.numpy as jnp
from jax import lax
from jax.experimental import pallas as pl
from jax.experimental.pallas import tpu as pltpu
from jax.experimental.pallas import tpu_sc as plsc

N = 2048
DIM = 1024
H = 16
DH = 64
BS = 4
NSEL = 4
WIN = 32
NB = N // BS
NW = N // WIN
NCH = 8
CH = N // NCH
SCALE = DH ** -0.5
NEG = -1e30


def _pre_body(lam_ref, x_ref, x0_ref, wq_ref, wk_ref, wv_ref, wg_ref,
              x1_ref, q_ref, k_ref, v_ref, g_ref):
    l0 = lam_ref[0]
    l1 = lam_ref[1]
    x1 = l0 * x_ref[...] + l1 * x0_ref[...]
    x1_ref[...] = x1
    ms = jnp.mean(x1 * x1, axis=-1, keepdims=True)
    xn = x1 * lax.rsqrt(ms + 1e-6)
    q_ref[...] = jnp.dot(xn, wq_ref[...], preferred_element_type=jnp.float32)
    k_ref[...] = jnp.dot(xn, wk_ref[...], preferred_element_type=jnp.float32)
    v_ref[...] = jnp.dot(xn, wv_ref[...], preferred_element_type=jnp.float32)
    g_ref[...] = jax.nn.sigmoid(
        jnp.dot(xn, wg_ref[...], preferred_element_type=jnp.float32))


def _pre(x, x0, lam, Wq, Wk, Wv, Wg):
    row = lambda c: (c, 0)
    full = lambda c: (0, 0)
    return pl.pallas_call(
        _pre_body,
        grid=(NCH,),
        in_specs=[
            pl.BlockSpec(memory_space=pltpu.SMEM),
            pl.BlockSpec((CH, DIM), row),
            pl.BlockSpec((CH, DIM), row),
            pl.BlockSpec((DIM, H * DH), full),
            pl.BlockSpec((DIM, H * DH), full),
            pl.BlockSpec((DIM, H * DH), full),
            pl.BlockSpec((DIM, H * 3), full),
        ],
        out_specs=[
            pl.BlockSpec((CH, DIM), row),
            pl.BlockSpec((CH, H * DH), row),
            pl.BlockSpec((CH, H * DH), row),
            pl.BlockSpec((CH, H * DH), row),
            pl.BlockSpec((CH, H * 3), row),
        ],
        out_shape=[
            jax.ShapeDtypeStruct((N, DIM), jnp.float32),
            jax.ShapeDtypeStruct((N, H * DH), jnp.float32),
            jax.ShapeDtypeStruct((N, H * DH), jnp.float32),
            jax.ShapeDtypeStruct((N, H * DH), jnp.float32),
            jax.ShapeDtypeStruct((N, H * 3), jnp.float32),
        ],
    )(lam, x, x0, Wq, Wk, Wv, Wg)


def _cmp_body(q_ref, kb_ref, vb_ref, pk_ref, pv_ref, wck_ref, wcv_ref,
              mk_ref, mv_ref, oc_ref, rows_ref, kpos_ref):
    h = pl.program_id(0)
    q = q_ref[0]
    kb = kb_ref[0] + pk_ref[...]
    vb = vb_ref[0] + pv_ref[...]
    ck = jnp.dot(kb, wck_ref[...], preferred_element_type=jnp.float32)
    cv = jnp.dot(vb, wcv_ref[...], preferred_element_type=jnp.float32)
    mk = mk_ref[0]
    mv = mv_ref[0]
    sim = lax.dot_general(q, ck, (((1,), (1,)), ((), ())),
                          preferred_element_type=jnp.float32) * SCALE
    sim_m = jnp.sum(q * mk, axis=-1, keepdims=True) * SCALE
    t = lax.broadcasted_iota(jnp.int32, (N, NB), 0)
    j = lax.broadcasted_iota(jnp.int32, (N, NB), 1)
    cmask = t >= (j + 1) * BS - 1
    sim = jnp.where(cmask, sim, NEG)
    m = jnp.maximum(jnp.max(sim, axis=-1, keepdims=True), sim_m)
    e = jnp.exp(sim - m)
    e = jnp.where(cmask, e, 0.0)
    em = jnp.exp(sim_m - m)
    denom = jnp.sum(e, axis=-1, keepdims=True) + em
    attn = e / denom
    oc_ref[0] = (jnp.dot(attn, cv, preferred_element_type=jnp.float32)
                 + (em / denom) * mv)
    own = j == t // BS
    cur = jnp.where(own, -jnp.inf, jnp.where(cmask, attn, -1.0))
    sels = [t[:, :1] // BS]
    for _ in range(NSEL - 1):
        mval = jnp.max(cur, axis=-1, keepdims=True)
        sel = jnp.min(jnp.where(cur == mval, j, NB), axis=-1, keepdims=True)
        sels.append(sel)
        cur = jnp.where(j == sel, -jnp.inf, cur)
    rows_ref[0] = jnp.concatenate(sels, axis=1) + h * NB
    kp = []
    for s in range(NSEL):
        for p in range(BS):
            kp.append(sels[s] * BS + p)
    kpos_ref[0] = jnp.concatenate(kp, axis=1)


def _cmp(q_t, kb4, vb4, pkf, pvf, Wck, Wcv, mem_k, mem_v):
    headN = lambda h: (h, 0, 0)
    full = lambda h: (0, 0)
    return pl.pallas_call(
        _cmp_body,
        grid=(H,),
        in_specs=[
            pl.BlockSpec((1, N, DH), headN),
            pl.BlockSpec((1, NB, BS * DH), headN),
            pl.BlockSpec((1, NB, BS * DH), headN),
            pl.BlockSpec((1, BS * DH), full),
            pl.BlockSpec((1, BS * DH), full),
            pl.BlockSpec((BS * DH, DH), full),
            pl.BlockSpec((BS * DH, DH), full),
            pl.BlockSpec((1, 1, DH), headN),
            pl.BlockSpec((1, 1, DH), headN),
        ],
        out_specs=[
            pl.BlockSpec((1, N, DH), headN),
            pl.BlockSpec((1, N, NSEL), headN),
            pl.BlockSpec((1, N, NSEL * BS), headN),
        ],
        out_shape=[
            jax.ShapeDtypeStruct((H, N, DH), jnp.float32),
            jax.ShapeDtypeStruct((H, N, NSEL), jnp.int32),
            jax.ShapeDtypeStruct((H, N, NSEL * BS), jnp.int32),
        ],
    )(q_t, kb4, vb4, pkf, pvf, Wck, Wcv, mem_k, mem_v)


GB = H * N * NSEL
ROWW = BS * DH
SC_CHUNK = 64


def _sc_gather(ktab, vtab, rows):
    info = plsc.get_sparse_core_info()
    nw = info.num_cores * info.num_subcores
    bpw = GB // nw
    nchunks = bpw // SC_CHUNK
    mesh = plsc.VectorSubcoreMesh(core_axis_name="c", subcore_axis_name="s")

    @functools.partial(
        pl.kernel, mesh=mesh,
        out_type=[jax.ShapeDtypeStruct((GB, ROWW), jnp.float32),
                  jax.ShapeDtypeStruct((GB, ROWW), jnp.float32)],
        scratch_types=[pltpu.VMEM((bpw,), jnp.int32),
                       pltpu.VMEM((SC_CHUNK, ROWW), jnp.float32),
                       pltpu.VMEM((SC_CHUNK, ROWW), jnp.float32),
                       pltpu.VMEM((SC_CHUNK, ROWW), jnp.float32),
                       pltpu.VMEM((SC_CHUNK, ROWW), jnp.float32),
                       pltpu.SemaphoreType.DMA,
                       pltpu.SemaphoreType.DMA],
    )
    def body(ktab_hbm, vtab_hbm, idx_hbm, gk_hbm, gv_hbm,
             idx_v, kbuf0, vbuf0, kbuf1, vbuf1, sem0, sem1):
        wid = lax.axis_index("s") * info.num_cores + lax.axis_index("c")
        base = wid * bpw
        pltpu.sync_copy(idx_hbm.at[pl.ds(base, bpw)], idx_v)
        bufs = [(kbuf0, vbuf0, sem0), (kbuf1, vbuf1, sem1)]
        handles = {}

        def issue(c):
            kb, vb, sm = bufs[c % 2]
            idx_slice = idx_v.at[pl.ds(c * SC_CHUNK, SC_CHUNK)]
            handles[c] = (pltpu.async_copy(ktab_hbm.at[idx_slice], kb, sm),
                          pltpu.async_copy(vtab_hbm.at[idx_slice], vb, sm))

        issue(0)
        for c in range(nchunks):
            if c + 1 < nchunks:
                issue(c + 1)
            kb, vb, _ = bufs[c % 2]
            hk, hv = handles.pop(c)
            hk.wait()
            hv.wait()
            pltpu.sync_copy(kb, gk_hbm.at[pl.ds(base + c * SC_CHUNK, SC_CHUNK)])
            pltpu.sync_copy(vb, gv_hbm.at[pl.ds(base + c * SC_CHUNK, SC_CHUNK)])

    return body(ktab, vtab, rows)


def _finw_body(q_ref, k_ref, v_ref, oc_ref, g_ref, out_ref):
    c = pl.program_id(1)
    q = q_ref[0]
    base = jnp.maximum(c * CH - WIN, 0)
    kb = k_ref[0, pl.ds(base, CH + WIN), :]
    vb = v_ref[0, pl.ds(base, CH + WIN), :]
    sw = lax.dot_general(q, kb, (((1,), (1,)), ((), ())),
                         preferred_element_type=jnp.float32) * SCALE
    sg = c * CH + lax.broadcasted_iota(jnp.int32, (CH, CH + WIN), 0)
    tg = base + lax.broadcasted_iota(jnp.int32, (CH, CH + WIN), 1)
    wm = (tg >= sg - (WIN - 1)) & (tg <= sg)
    sw = jnp.where(wm, sw, NEG)
    mw = jnp.max(sw, axis=-1, keepdims=True)
    ew = jnp.exp(sw - mw)
    ew = jnp.where(wm, ew, 0.0)
    pw = ew / jnp.sum(ew, axis=-1, keepdims=True)
    out_w = jnp.dot(pw, vb, preferred_element_type=jnp.float32)
    g = g_ref[0]
    out_ref[0] = g[:, 0:1] * oc_ref[0] + g[:, 2:3] * out_w


def _finw(q_t, k_t, v_t, out_c, g_t):
    hc = lambda h, c: (h, c, 0)
    hfull = lambda h, c: (h, 0, 0)
    return pl.pallas_call(
        _finw_body,
        grid=(H, NCH),
        in_specs=[
            pl.BlockSpec((1, CH, DH), hc),
            pl.BlockSpec((1, N, DH), hfull),
            pl.BlockSpec((1, N, DH), hfull),
            pl.BlockSpec((1, CH, DH), hc),
            pl.BlockSpec((1, CH, 3), hc),
        ],
        out_specs=pl.BlockSpec((1, CH, DH), hc),
        out_shape=jax.ShapeDtypeStruct((H, N, DH), jnp.float32),
    )(q_t, k_t, v_t, out_c, g_t)


def _finf_body(q_ref, gk_ref, gv_ref, kpos_ref, g_ref, part_ref, out_ref):
    c = pl.program_id(1)
    q = q_ref[0]
    gk = gk_ref[0]
    gv = gv_ref[0]
    kpos = kpos_ref[0]
    qpos = c * CH + lax.broadcasted_iota(jnp.int32, (CH, NSEL * BS), 0)
    simf = jnp.sum(gk * q[:, None, :], axis=-1) * SCALE
    simf = jnp.where(kpos <= qpos, simf, NEG)
    mf = jnp.max(simf, axis=-1, keepdims=True)
    ef = jnp.exp(simf - mf)
    pf = ef / jnp.sum(ef, axis=-1, keepdims=True)
    out_f = jnp.sum(pf[:, :, None] * gv, axis=1)
    g = g_ref[0]
    out_ref[0] = part_ref[0] + g[:, 1:2] * out_f


def _finf(q_t, gk4, gv4, kpos, g_t, part):
    hc = lambda h, c: (h, c, 0)
    hc4 = lambda h, c: (h, c, 0, 0)
    return pl.pallas_call(
        _finf_body,
        grid=(H, NCH),
        in_specs=[
            pl.BlockSpec((1, CH, DH), hc),
            pl.BlockSpec((1, CH, NSEL * BS, DH), hc4),
            pl.BlockSpec((1, CH, NSEL * BS, DH), hc4),
            pl.BlockSpec((1, CH, NSEL * BS), hc),
            pl.BlockSpec((1, CH, 3), hc),
            pl.BlockSpec((1, CH, DH), hc),
        ],
        out_specs=pl.BlockSpec((1, CH, DH), hc),
        out_shape=jax.ShapeDtypeStruct((H, N, DH), jnp.float32),
    )(q_t, gk4, gv4, kpos, g_t, part)


def _mlp_body(x1_ref, att_ref, wo_ref, w1_ref, w2_ref, y_ref):
    att16 = att_ref[...].astype(jnp.bfloat16)
    x2 = x1_ref[...] + jnp.dot(att16, wo_ref[...].astype(jnp.bfloat16),
                               preferred_element_type=jnp.float32)
    ms = jnp.mean(x2 * x2, axis=-1, keepdims=True)
    xn = x2 * lax.rsqrt(ms + 1e-6)
    hh = jnp.maximum(jnp.dot(xn.astype(jnp.bfloat16),
                             w1_ref[...].astype(jnp.bfloat16),
                             preferred_element_type=jnp.float32), 0.0)
    hh = hh * hh
    y_ref[...] = x2 + jnp.dot(hh.astype(jnp.bfloat16),
                              w2_ref[...].astype(jnp.bfloat16),
                              preferred_element_type=jnp.float32)


def _mlp(x1, att, Wo, W1, W2):
    row = lambda c: (c, 0)
    full = lambda c: (0, 0)
    return pl.pallas_call(
        _mlp_body,
        grid=(NCH,),
        in_specs=[
            pl.BlockSpec((CH, DIM), row),
            pl.BlockSpec((CH, DIM), row),
            pl.BlockSpec((DIM, DIM), full),
            pl.BlockSpec((DIM, 4 * DIM), full),
            pl.BlockSpec((4 * DIM, DIM), full),
        ],
        out_specs=pl.BlockSpec((CH, DIM), row),
        out_shape=jax.ShapeDtypeStruct((N, DIM), jnp.float32),
    )(x1, att, Wo, W1, W2)


def kernel(x, ve, x0, block_mask, lambdas, Wq, Wk, Wv, Wo, Wg, pos_k, pos_v,
           Wck, Wcv, mem_k, mem_v, W1, W2):
    xr = x.reshape(N, DIM)
    x0r = x0.reshape(N, DIM)
    x1, q, k, v, g48 = _pre(xr, x0r, lambdas, Wq, Wk, Wv, Wg)
    q_t = q.reshape(N, H, DH).transpose(1, 0, 2)
    k_t = k.reshape(N, H, DH).transpose(1, 0, 2)
    v_t = v.reshape(N, H, DH).transpose(1, 0, 2)
    g_t = g48.reshape(N, H, 3).transpose(1, 0, 2)
    kb4 = k_t.reshape(H, NB, BS * DH)
    vb4 = v_t.reshape(H, NB, BS * DH)
    pkf = pos_k.reshape(1, BS * DH)
    pvf = pos_v.reshape(1, BS * DH)
    out_c, rows, kpos = _cmp(q_t, kb4, vb4, pkf, pvf, Wck, Wcv, mem_k, mem_v)
    ktab = k_t.reshape(H * NB, BS * DH)
    vtab = v_t.reshape(H * NB, BS * DH)
    rflat = rows.reshape(-1)
    gk, gv = _sc_gather(ktab, vtab, rflat)
    gk4 = gk.reshape(H, N, NSEL * BS, DH)
    gv4 = gv.reshape(H, N, NSEL * BS, DH)
    part = _finw(q_t, k_t, v_t, out_c, g_t)
    att = _finf(q_t, gk4, gv4, kpos, g_t, part)
    att2 = att.transpose(1, 0, 2).reshape(N, H * DH)
    y = _mlp(x1, att2, Wo, W1, W2)
    return y.reshape(1, N, DIM)
